# baseline (speedup 1.0000x reference)
import functools

import jax
import jax.numpy as jnp
from jax import lax
from jax.experimental import pallas as pl
from jax.experimental.pallas import tpu as pltpu
from jax.experimental.pallas import tpu_sc as plsc

_V = 1000
_D = 1000
_BATCH = 4096
_HIST = 50
_NVT = 125
_NBT = 32
_NC = 2
_NS = 16
_COLS = 64

_mesh = plsc.VectorSubcoreMesh(core_axis_name="c", subcore_axis_name="s")


@functools.partial(
    pl.kernel,
    mesh=_mesh,
    out_type=jax.ShapeDtypeStruct((_HIST, _NVT, _NBT, 8, 128), jnp.float32),
    scratch_types=[
        pltpu.VMEM((_COLS * _V,), jnp.float32),
        pltpu.VMEM((2, _BATCH), jnp.int32),
        pltpu.VMEM((2, 8, 2, 8, 128), jnp.float32),
        pltpu.SemaphoreType.DMA,
        pltpu.SemaphoreType.DMA,
        pltpu.SemaphoreType.DMA,
    ],
    compiler_params=pltpu.CompilerParams(
        use_tc_tiling_on_sc=False, needs_layout_passes=False
    ),
)
def _sc_tgather(table_hbm, xt_hbm, out_hbm, tab_v, idx_v, stage_v,
                isem, asem, bsem):
    c = lax.axis_index("c")
    s = lax.axis_index("s")
    vt0 = s * _NVT // _NS
    nvt = (s + 1) * _NVT // _NS - vt0
    has8 = nvt == 8

    pltpu.sync_copy(table_hbm.at[pl.ds(vt0 * 8 * _V, _COLS * _V)], tab_v)

    def idx_start(l, slot):
        pltpu.async_copy(xt_hbm.at[l], idx_v.at[slot], isem)

    def idx_wait(slot):
        pltpu.make_async_copy(xt_hbm.at[0], idx_v.at[slot], isem).wait()

    def write_start(l, bt0, oslot):
        pltpu.async_copy(
            stage_v.at[oslot, pl.ds(0, 7)],
            out_hbm.at[l, pl.ds(vt0, 7), pl.ds(bt0, 2)],
            asem,
        )

        @pl.when(has8)
        def _():
            pltpu.async_copy(
                stage_v.at[oslot, 7],
                out_hbm.at[l, vt0 + 7, pl.ds(bt0, 2)],
                bsem,
            )

    def write_wait():
        pltpu.make_async_copy(
            stage_v.at[0, pl.ds(0, 7)],
            out_hbm.at[0, pl.ds(0, 7), pl.ds(0, 2)],
            asem,
        ).wait()

        @pl.when(has8)
        def _():
            pltpu.make_async_copy(
                stage_v.at[0, 7], out_hbm.at[0, 0, pl.ds(0, 2)], bsem
            ).wait()

    idx_start(0, 0)

    def l_body(l, carry):
        lslot = l % 2
        idx_wait(lslot)

        @pl.when(l + 1 < _HIST)
        def _():
            idx_start(l + 1, 1 - lslot)

        def j_body(j, carry2):
            bt0 = c * 16 + j * 2
            oslot = j % 2

            @pl.when((l > 0) | (j >= 2))
            def _():
                write_wait()

            rows = [
                idx_v.at[lslot][pl.ds(bt0 * 128 + g * 16, 16)]
                for g in range(16)
            ]

            def do_vt(vt):
                for vs in range(8):
                    sub = tab_v.at[pl.ds((vt * 8 + vs) * _V, _V)]
                    vals = [
                        plsc.load_gather(sub, [rows[g]]) for g in range(16)
                    ]
                    for btl in range(2):
                        for g in range(8):
                            stage_v[
                                oslot, vt, btl, vs, pl.ds(g * 16, 16)
                            ] = vals[btl * 8 + g]

            for vt in range(7):
                do_vt(vt)

            @pl.when(has8)
            def _():
                do_vt(7)

            write_start(l, bt0, oslot)
            return carry2

        lax.fori_loop(0, 8, j_body, 0)
        return carry

    lax.fori_loop(0, _HIST, l_body, 0)
    write_wait()
    write_wait()


def kernel(x, table):
    xt = jnp.transpose(x.astype(jnp.int32))
    tflat = jnp.transpose(table).reshape(-1)
    out5d = _sc_tgather(tflat, xt)
    return jnp.transpose(out5d, (2, 4, 0, 1, 3)).reshape(_BATCH, _HIST, _D)


# speedup vs baseline: 2.4054x; 2.4054x over previous
import functools

import jax
import jax.numpy as jnp
from jax import lax
from jax.experimental import pallas as pl
from jax.experimental.pallas import tpu as pltpu
from jax.experimental.pallas import tpu_sc as plsc

_V = 1000
_D = 1000
_BATCH = 4096
_HIST = 50
_NVT = 125
_NBT = 32
_NC = 2
_NS = 16
_COLS = 64

_mesh = plsc.VectorSubcoreMesh(core_axis_name="c", subcore_axis_name="s")


@functools.partial(
    pl.kernel,
    mesh=_mesh,
    out_type=jax.ShapeDtypeStruct((_HIST, _NVT, _NBT, 8, 128), jnp.float32),
    scratch_types=[
        pltpu.VMEM((_COLS * _V,), jnp.float32),
        pltpu.VMEM((2, _BATCH), jnp.int32),
        pltpu.VMEM((2, 8, 8, 128), jnp.float32),
        pltpu.SemaphoreType.DMA,
        pltpu.SemaphoreType.DMA,
        pltpu.SemaphoreType.DMA,
    ],
    compiler_params=pltpu.CompilerParams(
        use_tc_tiling_on_sc=False, needs_layout_passes=False
    ),
)
def _sc_tgather(table_hbm, xt_hbm, out_hbm, tab_v, idx_v, stage_v,
                isem, asem, bsem):
    c = lax.axis_index("c")
    s = lax.axis_index("s")
    vt0 = s * _NVT // _NS
    nvt = (s + 1) * _NVT // _NS - vt0
    has8 = nvt == 8

    pltpu.sync_copy(table_hbm.at[pl.ds(vt0 * 8 * _V, _COLS * _V)], tab_v)

    def idx_start(l, slot):
        pltpu.async_copy(xt_hbm.at[l], idx_v.at[slot], isem)

    def idx_wait(slot):
        pltpu.make_async_copy(xt_hbm.at[0], idx_v.at[slot], isem).wait()

    def write_start(l, bt, oslot):
        pltpu.async_copy(
            stage_v.at[oslot, pl.ds(0, 7)],
            out_hbm.at[l, pl.ds(vt0, 7), bt],
            asem,
        )

        @pl.when(has8)
        def _():
            pltpu.async_copy(
                stage_v.at[oslot, 7], out_hbm.at[l, vt0 + 7, bt], bsem
            )

    def write_wait():
        pltpu.make_async_copy(
            stage_v.at[0, pl.ds(0, 7)], out_hbm.at[0, pl.ds(0, 7), 0], asem
        ).wait()

        @pl.when(has8)
        def _():
            pltpu.make_async_copy(
                stage_v.at[0, 7], out_hbm.at[0, 0, 0], bsem
            ).wait()

    idx_start(0, 0)

    def l_body(l, carry):
        lslot = l % 2
        idx_wait(lslot)

        @pl.when(l + 1 < _HIST)
        def _():
            idx_start(l + 1, 1 - lslot)

        def j_body(j, carry2):
            bt = j * _NC + c
            oslot = j % 2

            @pl.when((l > 0) | (j >= 2))
            def _():
                write_wait()

            rows = [
                idx_v.at[lslot][pl.ds(bt * 128 + g * 16, 16)] for g in range(8)
            ]

            def do_vt(vt):
                for vs in range(8):
                    sub = tab_v.at[pl.ds((vt * 8 + vs) * _V, _V)]
                    vals = [plsc.load_gather(sub, [rows[g]]) for g in range(8)]
                    for g in range(8):
                        stage_v[oslot, vt, vs, pl.ds(g * 16, 16)] = vals[g]

            for vt in range(7):
                do_vt(vt)

            @pl.when(has8)
            def _():
                do_vt(7)

            write_start(l, bt, oslot)
            return carry2

        lax.fori_loop(0, 16, j_body, 0)
        return carry

    lax.fori_loop(0, _HIST, l_body, 0)
    write_wait()
    write_wait()


def kernel(x, table):
    xt = jnp.transpose(x.astype(jnp.int32))
    tflat = jnp.transpose(table).reshape(-1)
    out5d = _sc_tgather(tflat, xt)
    return jnp.transpose(out5d, (2, 4, 0, 1, 3)).reshape(_BATCH, _HIST, _D)
